# baseline (speedup 1.0000x reference)
import functools
import math

import jax
import jax.numpy as jnp
from jax import lax
from jax.experimental import pallas as pl
from jax.experimental.pallas import tpu as pltpu
from jax.experimental.pallas import tpu_sc as plsc

NC = 2
NS = 16


def _rup(v, m):
    return ((v + m - 1) // m) * m


def _gelu(v):
    return 0.5 * v * (1.0 + lax.erf(v * jnp.float32(0.7071067811865476)))


def _ln(v, g, b):
    m = jnp.mean(v, axis=-1, keepdims=True)
    c = v - m
    var = jnp.mean(c * c, axis=-1, keepdims=True)
    return c * lax.rsqrt(var + jnp.float32(1e-5)) * g + b


def _impl(H, W, x, t, norm, flat_idx, conv_w, ln1_g, ln1_b, ln2_g, ln2_b,
          tmlp1_w, tmlp1_b, tmlp2_w, tmlp2_b, mlp_w1, mlp_b1, mlp_w2, mlp_b2):
    N, C = x.shape
    B = t.shape[0]
    K = conv_w.shape[0]
    R = K // 2
    HW = H * W
    Wp = W + 2 * R
    acc_need = (H - 1) * Wp + W
    BLK = min(256, _rup(acc_need, 8))
    nblk = -(-acc_need // BLK)
    ACC_B = nblk * BLK
    omax = (K - 1) * (Wp + 1)
    PB = _rup(ACC_B + omax, 8)
    NB = N // B
    RB = 512 if NB % 512 == 0 else NB
    nrb = N // RB
    TRASH = B * PB
    GR = _rup(B * PB + 8, NS * 8)
    NCH = N // 128

    f32 = jnp.float32
    i32 = jnp.int32
    w49 = conv_w.reshape(K * K, C, C).astype(jnp.bfloat16)
    vspec = pl.BlockSpec(memory_space=pltpu.VMEM)

    def a_body(x_r, t_r, vcol_r, v2d_r, ln1g_r, ln1b_r, t1w_r, t1b_r,
               h_r, si_r, gi_r):
        tg = _gelu(t_r[...])
        tt1 = jnp.dot(tg, t1w_r[...], preferred_element_type=f32) + t1b_r[...]
        ln1g = ln1g_r[...]
        ln1b = ln1b_r[...]
        for rb in range(nrb):
            sl = pl.ds(rb * RB, RB)
            bi = (rb * RB) // NB
            f = _ln(x_r[sl, :], ln1g, ln1b)
            h_r[sl, :] = f * (1.0 + tt1[bi:bi + 1, :C]) + tt1[bi:bi + 1, C:]

        vcol = vcol_r[...]
        iglob = lax.broadcasted_iota(i32, (N, 128), 0)
        lane = lax.broadcasted_iota(i32, (N, 128), 1)
        dup = jnp.zeros((N, 1), i32)
        for jc in range(N // 128):
            cj = v2d_r[jc:jc + 1, :]
            eq = (vcol == cj) & ((jc * 128 + lane) > iglob)
            dup = jnp.maximum(dup, jnp.max(eq.astype(i32), axis=1,
                                           keepdims=True))
        b = vcol // HW
        r = vcol % HW
        y = r // W
        xx = r % W
        si = jnp.where(dup > 0, TRASH, b * PB + (y + R) * Wp + xx + R)
        si_r[...] = si
        gi_r[...] = b * ACC_B + y * Wp + xx

    h, si, gi = pl.pallas_call(
        a_body,
        out_shape=[jax.ShapeDtypeStruct((N, C), f32),
                   jax.ShapeDtypeStruct((N, 1), i32),
                   jax.ShapeDtypeStruct((N, 1), i32)],
        in_specs=[vspec] * 8,
        out_specs=[vspec] * 3,
    )(x, t, flat_idx.astype(i32).reshape(N, 1),
      flat_idx.astype(i32).reshape(N // 128, 128),
      ln1_g.reshape(1, C), ln1_b.reshape(1, C),
      tmlp1_w, tmlp1_b.reshape(1, 2 * C))
    si = si.reshape(N)
    gi = gi.reshape(N)

    mesh = plsc.VectorSubcoreMesh(core_axis_name="c", subcore_axis_name="s")
    rows_per_tile = N // NS
    nchunk = rows_per_tile // 128
    zr = GR // NS

    @functools.partial(
        pl.kernel, mesh=mesh,
        out_type=jax.ShapeDtypeStruct((GR, C), f32),
        scratch_types=[pltpu.VMEM((128,), i32),
                       pltpu.VMEM((128, C), f32),
                       pltpu.SemaphoreType.DMA],
    )
    def scat_k(zin_h, h_h, si_h, grid_h, idx_v, rows_v, sem):
        cid = lax.axis_index("c")
        sid = lax.axis_index("s")

        @pl.when(cid == 0)
        def _():
            pltpu.sync_copy(zin_h.at[pl.ds(sid * zr, zr)],
                            grid_h.at[pl.ds(sid * zr, zr)])
            plsc.subcore_barrier()
            for ch in range(nchunk):
                base = sid * rows_per_tile + ch * 128
                pltpu.sync_copy(si_h.at[pl.ds(base, 128)], idx_v)
                pltpu.sync_copy(h_h.at[pl.ds(base, 128)], rows_v)
                pltpu.async_copy(rows_v, grid_h.at[idx_v], sem).wait()

    grid = scat_k(jnp.zeros((GR, C), f32), h, si)

    def c_body(grid_r, w_r, acc_r):
        def conv_blk(m, c):
            b = m // nblk
            g0 = (m % nblk) * BLK

            def tap(tp, acc):
                o = (tp // K) * Wp + (tp % K)
                src = grid_r[pl.ds(b * PB + g0 + o, BLK), :]
                return acc + jnp.dot(src.astype(jnp.bfloat16), w_r[tp],
                                     preferred_element_type=f32)

            accv = lax.fori_loop(0, K * K, tap, jnp.zeros((BLK, C), f32))
            acc_r[pl.ds(b * ACC_B + g0, BLK), :] = accv
            return c

        lax.fori_loop(0, B * nblk, conv_blk, 0)

    acc = pl.pallas_call(
        c_body,
        out_shape=jax.ShapeDtypeStruct((B * ACC_B, C), f32),
        in_specs=[vspec] * 2,
        out_specs=vspec,
    )(grid, w49)

    @functools.partial(
        pl.kernel, mesh=mesh,
        out_type=jax.ShapeDtypeStruct((N, C), f32),
        scratch_types=[pltpu.VMEM((128,), i32),
                       pltpu.VMEM((128, C), f32),
                       pltpu.SemaphoreType.DMA],
    )
    def gath_k(acc_h, gi_h, out_h, idx_v, rows_v, sem):
        wid = lax.axis_index("s") * NC + lax.axis_index("c")
        base = wid * (N // (NC * NS))
        pltpu.sync_copy(gi_h.at[pl.ds(base, 128)], idx_v)
        pltpu.async_copy(acc_h.at[idx_v], rows_v, sem).wait()
        pltpu.sync_copy(rows_v, out_h.at[pl.ds(base, 128)])

    hc = gath_k(acc, gi)

    def e_body(x_r, hc_r, norm_r, t_r, ln2g_r, ln2b_r, t2w_r, t2b_r,
               m1w_r, m1b_r, m2w_r, m2b_r, out_r):
        tg = _gelu(t_r[...])
        tt2 = jnp.dot(tg, t2w_r[...], preferred_element_type=f32) + t2b_r[...]
        ln2g = ln2g_r[...]
        ln2b = ln2b_r[...]
        m1w = m1w_r[...]
        m1b = m1b_r[...]
        m2w = m2w_r[...]
        m2b = m2b_r[...]
        for rb in range(nrb):
            sl = pl.ds(rb * RB, RB)
            bi = (rb * RB) // NB
            x1 = x_r[sl, :] + hc_r[sl, :] / norm_r[sl, :]
            f2 = _ln(x1, ln2g, ln2b)
            h2 = f2 * (1.0 + tt2[bi:bi + 1, :C]) + tt2[bi:bi + 1, C:]
            hid = _gelu(jnp.dot(h2, m1w, preferred_element_type=f32) + m1b)
            o = jnp.dot(hid, m2w, preferred_element_type=f32) + m2b
            out_r[sl, :] = x1 + o

    return pl.pallas_call(
        e_body,
        out_shape=jax.ShapeDtypeStruct((N, C), f32),
        in_specs=[vspec] * 12,
        out_specs=vspec,
    )(x, hc, norm, t, ln2_g.reshape(1, C), ln2_b.reshape(1, C),
      tmlp2_w, tmlp2_b.reshape(1, 2 * C),
      mlp_w1, mlp_b1.reshape(1, 2 * C), mlp_w2, mlp_b2.reshape(1, C))


def kernel(x, t, norm, flat_idx, conv_w, ln1_g, ln1_b, ln2_g, ln2_b,
           tmlp1_w, tmlp1_b, tmlp2_w, tmlp2_b, mlp_w1, mlp_b1, mlp_w2, mlp_b2):
    return _impl(64, 64, x, t, norm, flat_idx, conv_w, ln1_g, ln1_b,
                 ln2_g, ln2_b, tmlp1_w, tmlp1_b, tmlp2_w, tmlp2_b,
                 mlp_w1, mlp_b1, mlp_w2, mlp_b2)


# speedup vs baseline: 2.7287x; 2.7287x over previous
import functools
import math

import jax
import jax.numpy as jnp
from jax import lax
from jax.experimental import pallas as pl
from jax.experimental.pallas import tpu as pltpu
from jax.experimental.pallas import tpu_sc as plsc

NC = 2
NS = 16


def _rup(v, m):
    return ((v + m - 1) // m) * m


def _gelu(v):
    return 0.5 * v * (1.0 + lax.erf(v * jnp.float32(0.7071067811865476)))


def _ln(v, g, b):
    m = jnp.mean(v, axis=-1, keepdims=True)
    c = v - m
    var = jnp.mean(c * c, axis=-1, keepdims=True)
    return c * lax.rsqrt(var + jnp.float32(1e-5)) * g + b


def _impl(H, W, x, t, norm, flat_idx, conv_w, ln1_g, ln1_b, ln2_g, ln2_b,
          tmlp1_w, tmlp1_b, tmlp2_w, tmlp2_b, mlp_w1, mlp_b1, mlp_w2, mlp_b2):
    N, C = x.shape
    B = t.shape[0]
    K = conv_w.shape[0]
    R = K // 2
    HW = H * W
    Wp = W + 2 * R
    acc_need = (H - 1) * Wp + W
    BLK = min(256, _rup(acc_need, 8))
    nblk = -(-acc_need // BLK)
    ACC_B = nblk * BLK
    omax = (K - 1) * (Wp + 1)
    PB = _rup(ACC_B + omax, 8)
    NB = N // B
    RB = 512 if NB % 512 == 0 else NB
    nrb = N // RB
    TRASH = B * PB
    GR = _rup(B * PB + 8, NS * 8)
    NCH = N // 128

    f32 = jnp.float32
    i32 = jnp.int32
    wky = conv_w.reshape(K, K * C, C).astype(jnp.bfloat16)
    vspec = pl.BlockSpec(memory_space=pltpu.VMEM)

    def a_body(x_r, t_r, vcol_r, v2d_r, ln1g_r, ln1b_r, t1w_r, t1b_r,
               h_r, si_r, gi_r):
        tg = _gelu(t_r[...])
        tt1 = jnp.dot(tg, t1w_r[...], preferred_element_type=f32) + t1b_r[...]
        ln1g = ln1g_r[...]
        ln1b = ln1b_r[...]
        for rb in range(nrb):
            sl = pl.ds(rb * RB, RB)
            bi = (rb * RB) // NB
            f = _ln(x_r[sl, :], ln1g, ln1b)
            h_r[sl, :] = f * (1.0 + tt1[bi:bi + 1, :C]) + tt1[bi:bi + 1, C:]

        vcol = vcol_r[...]
        iglob = lax.broadcasted_iota(i32, (N, 128), 0)
        lane = lax.broadcasted_iota(i32, (N, 128), 1)
        dup = jnp.zeros((N, 1), i32)
        for jc in range(N // 128):
            cj = v2d_r[jc:jc + 1, :]
            eq = (vcol == cj) & ((jc * 128 + lane) > iglob)
            dup = jnp.maximum(dup, jnp.max(eq.astype(i32), axis=1,
                                           keepdims=True))
        b = vcol // HW
        r = vcol % HW
        y = r // W
        xx = r % W
        si = jnp.where(dup > 0, TRASH, b * PB + (y + R) * Wp + xx + R)
        si_r[...] = si
        gi_r[...] = b * ACC_B + y * Wp + xx

    h, si, gi = pl.pallas_call(
        a_body,
        out_shape=[jax.ShapeDtypeStruct((N, C), f32),
                   jax.ShapeDtypeStruct((N, 1), i32),
                   jax.ShapeDtypeStruct((N, 1), i32)],
        in_specs=[vspec] * 8,
        out_specs=[vspec] * 3,
    )(x, t, flat_idx.astype(i32).reshape(N, 1),
      flat_idx.astype(i32).reshape(N // 128, 128),
      ln1_g.reshape(1, C), ln1_b.reshape(1, C),
      tmlp1_w, tmlp1_b.reshape(1, 2 * C))
    si = si.reshape(N)
    gi = gi.reshape(N)

    mesh = plsc.VectorSubcoreMesh(core_axis_name="c", subcore_axis_name="s")
    rows_per_tile = N // NS
    nchunk = rows_per_tile // 128
    zr = GR // NS

    @functools.partial(
        pl.kernel, mesh=mesh,
        out_type=jax.ShapeDtypeStruct((GR, C), f32),
        scratch_types=[pltpu.VMEM((128,), i32),
                       pltpu.VMEM((128, C), f32),
                       pltpu.SemaphoreType.DMA],
    )
    def scat_k(zin_h, h_h, si_h, grid_h, idx_v, rows_v, sem):
        cid = lax.axis_index("c")
        sid = lax.axis_index("s")

        @pl.when(cid == 0)
        def _():
            pltpu.sync_copy(zin_h, rows_v)
            zoff = 0
            while zoff < zr:
                zc = min(128, zr - zoff)
                pltpu.sync_copy(rows_v.at[pl.ds(0, zc)],
                                grid_h.at[pl.ds(sid * zr + zoff, zc)])
                zoff += zc
            plsc.subcore_barrier()
            for ch in range(nchunk):
                base = sid * rows_per_tile + ch * 128
                pltpu.sync_copy(si_h.at[pl.ds(base, 128)], idx_v)
                pltpu.sync_copy(h_h.at[pl.ds(base, 128)], rows_v)
                pltpu.async_copy(rows_v, grid_h.at[idx_v], sem).wait()

    grid = scat_k(jnp.zeros((128, C), f32), h, si)

    def c_body(grid_r, w_r, acc_r):
        def conv_blk(m, c):
            b = m // nblk
            g0 = (m % nblk) * BLK

            def ky_step(ky, acc):
                base = b * PB + g0 + ky * Wp
                src = jnp.concatenate(
                    [grid_r[pl.ds(base + kx, BLK), :] for kx in range(K)],
                    axis=1).astype(jnp.bfloat16)
                return acc + jnp.dot(src, w_r[ky],
                                     preferred_element_type=f32)

            accv = lax.fori_loop(0, K, ky_step, jnp.zeros((BLK, C), f32))
            acc_r[pl.ds(b * ACC_B + g0, BLK), :] = accv
            return c

        lax.fori_loop(0, B * nblk, conv_blk, 0)

    acc = pl.pallas_call(
        c_body,
        out_shape=jax.ShapeDtypeStruct((B * ACC_B, C), f32),
        in_specs=[vspec] * 2,
        out_specs=vspec,
    )(grid, wky)

    @functools.partial(
        pl.kernel, mesh=mesh,
        out_type=jax.ShapeDtypeStruct((N, C), f32),
        scratch_types=[pltpu.VMEM((128,), i32),
                       pltpu.VMEM((128, C), f32),
                       pltpu.SemaphoreType.DMA],
    )
    def gath_k(acc_h, gi_h, out_h, idx_v, rows_v, sem):
        wid = lax.axis_index("s") * NC + lax.axis_index("c")
        base = wid * (N // (NC * NS))
        pltpu.sync_copy(gi_h.at[pl.ds(base, 128)], idx_v)
        pltpu.async_copy(acc_h.at[idx_v], rows_v, sem).wait()
        pltpu.sync_copy(rows_v, out_h.at[pl.ds(base, 128)])

    hc = gath_k(acc, gi)

    def e_body(x_r, hc_r, norm_r, t_r, ln2g_r, ln2b_r, t2w_r, t2b_r,
               m1w_r, m1b_r, m2w_r, m2b_r, out_r):
        tg = _gelu(t_r[...])
        tt2 = jnp.dot(tg, t2w_r[...], preferred_element_type=f32) + t2b_r[...]
        ln2g = ln2g_r[...]
        ln2b = ln2b_r[...]
        m1w = m1w_r[...]
        m1b = m1b_r[...]
        m2w = m2w_r[...]
        m2b = m2b_r[...]
        for rb in range(nrb):
            sl = pl.ds(rb * RB, RB)
            bi = (rb * RB) // NB
            x1 = x_r[sl, :] + hc_r[sl, :] / norm_r[sl, :]
            f2 = _ln(x1, ln2g, ln2b)
            h2 = f2 * (1.0 + tt2[bi:bi + 1, :C]) + tt2[bi:bi + 1, C:]
            hid = _gelu(jnp.dot(h2, m1w, preferred_element_type=f32) + m1b)
            o = jnp.dot(hid, m2w, preferred_element_type=f32) + m2b
            out_r[sl, :] = x1 + o

    return pl.pallas_call(
        e_body,
        out_shape=jax.ShapeDtypeStruct((N, C), f32),
        in_specs=[vspec] * 12,
        out_specs=vspec,
    )(x, hc, norm, t, ln2_g.reshape(1, C), ln2_b.reshape(1, C),
      tmlp2_w, tmlp2_b.reshape(1, 2 * C),
      mlp_w1, mlp_b1.reshape(1, 2 * C), mlp_w2, mlp_b2.reshape(1, C))


def kernel(x, t, norm, flat_idx, conv_w, ln1_g, ln1_b, ln2_g, ln2_b,
           tmlp1_w, tmlp1_b, tmlp2_w, tmlp2_b, mlp_w1, mlp_b1, mlp_w2, mlp_b2):
    return _impl(64, 64, x, t, norm, flat_idx, conv_w, ln1_g, ln1_b,
                 ln2_g, ln2_b, tmlp1_w, tmlp1_b, tmlp2_w, tmlp2_b,
                 mlp_w1, mlp_b1, mlp_w2, mlp_b2)


# speedup vs baseline: 2.7765x; 1.0175x over previous
import functools
import math

import jax
import jax.numpy as jnp
from jax import lax
from jax.experimental import pallas as pl
from jax.experimental.pallas import tpu as pltpu
from jax.experimental.pallas import tpu_sc as plsc

NC = 2
NS = 16


def _rup(v, m):
    return ((v + m - 1) // m) * m


def _gelu(v):
    return 0.5 * v * (1.0 + lax.erf(v * jnp.float32(0.7071067811865476)))


def _ln(v, g, b):
    m = jnp.mean(v, axis=-1, keepdims=True)
    c = v - m
    var = jnp.mean(c * c, axis=-1, keepdims=True)
    return c * lax.rsqrt(var + jnp.float32(1e-5)) * g + b


def _impl(H, W, x, t, norm, flat_idx, conv_w, ln1_g, ln1_b, ln2_g, ln2_b,
          tmlp1_w, tmlp1_b, tmlp2_w, tmlp2_b, mlp_w1, mlp_b1, mlp_w2, mlp_b2):
    N, C = x.shape
    B = t.shape[0]
    K = conv_w.shape[0]
    R = K // 2
    HW = H * W
    Wp = W + 2 * R
    acc_need = (H - 1) * Wp + W
    BLK = min(256, _rup(acc_need, 8))
    nblk = -(-acc_need // BLK)
    ACC_B = nblk * BLK
    omax = (K - 1) * (Wp + 1)
    PB = _rup(ACC_B + omax, 8)
    NB = N // B
    RB = 512 if NB % 512 == 0 else NB
    nrb = N // RB
    TRASH = B * PB
    GR = _rup(B * PB + 8, NS * 8)
    NCH = N // 128

    f32 = jnp.float32
    i32 = jnp.int32
    wky = conv_w.reshape(K, K * C, C).astype(jnp.bfloat16)
    vspec = pl.BlockSpec(memory_space=pltpu.VMEM)

    def a_body(x_r, t_r, vcol_r, v2d_r, ln1g_r, ln1b_r, t1w_r, t1b_r,
               h_r, si_r, gi_r):
        tg = _gelu(t_r[...])
        tt1 = jnp.dot(tg, t1w_r[...], preferred_element_type=f32) + t1b_r[...]
        ln1g = ln1g_r[...]
        ln1b = ln1b_r[...]
        for rb in range(nrb):
            sl = pl.ds(rb * RB, RB)
            bi = (rb * RB) // NB
            f = _ln(x_r[sl, :], ln1g, ln1b)
            h_r[sl, :] = f * (1.0 + tt1[bi:bi + 1, :C]) + tt1[bi:bi + 1, C:]

        vcol = vcol_r[...]
        iglob = lax.broadcasted_iota(i32, (N, 128), 0)
        lane = lax.broadcasted_iota(i32, (N, 128), 1)
        dup = jnp.zeros((N, 1), i32)
        for jc in range(N // 128):
            cj = v2d_r[jc:jc + 1, :]
            eq = (vcol == cj) & ((jc * 128 + lane) > iglob)
            dup = jnp.maximum(dup, jnp.max(eq.astype(i32), axis=1,
                                           keepdims=True))
        b = vcol // HW
        r = vcol % HW
        y = r // W
        xx = r % W
        si = jnp.where(dup > 0, TRASH, b * PB + (y + R) * Wp + xx + R)
        si_r[...] = si
        gi_r[...] = b * ACC_B + y * Wp + xx

    h, si, gi = pl.pallas_call(
        a_body,
        out_shape=[jax.ShapeDtypeStruct((N, C), f32),
                   jax.ShapeDtypeStruct((N, 1), i32),
                   jax.ShapeDtypeStruct((N, 1), i32)],
        in_specs=[vspec] * 8,
        out_specs=[vspec] * 3,
    )(x, t, flat_idx.astype(i32).reshape(N, 1),
      flat_idx.astype(i32).reshape(N // 128, 128),
      ln1_g.reshape(1, C), ln1_b.reshape(1, C),
      tmlp1_w, tmlp1_b.reshape(1, 2 * C))
    si = si.reshape(N)
    gi = gi.reshape(N)

    mesh = plsc.VectorSubcoreMesh(core_axis_name="c", subcore_axis_name="s")
    rows_per_tile = N // NS
    nchunk = rows_per_tile // 128
    zr = GR // NS

    @functools.partial(
        pl.kernel, mesh=mesh,
        out_type=jax.ShapeDtypeStruct((GR, C), f32),
        scratch_types=[[pltpu.VMEM((128,), i32) for _ in range(nchunk)],
                       [pltpu.VMEM((128, C), f32) for _ in range(nchunk)],
                       pltpu.VMEM((128, C), f32),
                       pltpu.SemaphoreType.DMA,
                       pltpu.SemaphoreType.DMA,
                       pltpu.SemaphoreType.DMA],
    )
    def scat_k(zin_h, h_h, si_h, grid_h, idx_v, rows_v, zbuf_v,
               zsem, lsem, ssem):
        cid = lax.axis_index("c")
        sid = lax.axis_index("s")

        @pl.when(cid == 0)
        def _():
            pltpu.sync_copy(zin_h, zbuf_v)
            zcp = []
            zoff = 0
            while zoff < zr:
                zc = min(128, zr - zoff)
                zcp.append(pltpu.async_copy(
                    zbuf_v.at[pl.ds(0, zc)],
                    grid_h.at[pl.ds(sid * zr + zoff, zc)], zsem))
                zoff += zc
            lcp = []
            for ch in range(nchunk):
                base = sid * rows_per_tile + ch * 128
                lcp.append(pltpu.async_copy(si_h.at[pl.ds(base, 128)],
                                            idx_v[ch], lsem))
                lcp.append(pltpu.async_copy(h_h.at[pl.ds(base, 128)],
                                            rows_v[ch], lsem))
            for cp in zcp:
                cp.wait()
            plsc.subcore_barrier()
            for cp in lcp:
                cp.wait()
            scp = [pltpu.async_copy(rows_v[ch], grid_h.at[idx_v[ch]], ssem)
                   for ch in range(nchunk)]
            for cp in scp:
                cp.wait()

    grid = scat_k(jnp.zeros((128, C), f32), h, si)

    def c_body(grid_r, w_r, acc_r):
        def conv_blk(m, c):
            b = m // nblk
            g0 = (m % nblk) * BLK

            def ky_step(ky, acc):
                base = b * PB + g0 + ky * Wp
                src = jnp.concatenate(
                    [grid_r[pl.ds(base + kx, BLK), :] for kx in range(K)],
                    axis=1).astype(jnp.bfloat16)
                return acc + jnp.dot(src, w_r[ky],
                                     preferred_element_type=f32)

            accv = lax.fori_loop(0, K, ky_step, jnp.zeros((BLK, C), f32))
            acc_r[pl.ds(b * ACC_B + g0, BLK), :] = accv
            return c

        lax.fori_loop(0, B * nblk, conv_blk, 0)

    acc = pl.pallas_call(
        c_body,
        out_shape=jax.ShapeDtypeStruct((B * ACC_B, C), f32),
        in_specs=[vspec] * 2,
        out_specs=vspec,
    )(grid, wky)

    @functools.partial(
        pl.kernel, mesh=mesh,
        out_type=jax.ShapeDtypeStruct((N, C), f32),
        scratch_types=[pltpu.VMEM((128,), i32),
                       pltpu.VMEM((128, C), f32),
                       pltpu.SemaphoreType.DMA],
    )
    def gath_k(acc_h, gi_h, out_h, idx_v, rows_v, sem):
        wid = lax.axis_index("s") * NC + lax.axis_index("c")
        base = wid * (N // (NC * NS))
        pltpu.sync_copy(gi_h.at[pl.ds(base, 128)], idx_v)
        pltpu.async_copy(acc_h.at[idx_v], rows_v, sem).wait()
        pltpu.sync_copy(rows_v, out_h.at[pl.ds(base, 128)])

    hc = gath_k(acc, gi)

    def e_body(x_r, hc_r, norm_r, t_r, ln2g_r, ln2b_r, t2w_r, t2b_r,
               m1w_r, m1b_r, m2w_r, m2b_r, out_r):
        tg = _gelu(t_r[...])
        tt2 = jnp.dot(tg, t2w_r[...], preferred_element_type=f32) + t2b_r[...]
        ln2g = ln2g_r[...]
        ln2b = ln2b_r[...]
        m1w = m1w_r[...]
        m1b = m1b_r[...]
        m2w = m2w_r[...]
        m2b = m2b_r[...]
        for rb in range(nrb):
            sl = pl.ds(rb * RB, RB)
            bi = (rb * RB) // NB
            x1 = x_r[sl, :] + hc_r[sl, :] / norm_r[sl, :]
            f2 = _ln(x1, ln2g, ln2b)
            h2 = f2 * (1.0 + tt2[bi:bi + 1, :C]) + tt2[bi:bi + 1, C:]
            hid = _gelu(jnp.dot(h2, m1w, preferred_element_type=f32) + m1b)
            o = jnp.dot(hid, m2w, preferred_element_type=f32) + m2b
            out_r[sl, :] = x1 + o

    return pl.pallas_call(
        e_body,
        out_shape=jax.ShapeDtypeStruct((N, C), f32),
        in_specs=[vspec] * 12,
        out_specs=vspec,
    )(x, hc, norm, t, ln2_g.reshape(1, C), ln2_b.reshape(1, C),
      tmlp2_w, tmlp2_b.reshape(1, 2 * C),
      mlp_w1, mlp_b1.reshape(1, 2 * C), mlp_w2, mlp_b2.reshape(1, C))


def kernel(x, t, norm, flat_idx, conv_w, ln1_g, ln1_b, ln2_g, ln2_b,
           tmlp1_w, tmlp1_b, tmlp2_w, tmlp2_b, mlp_w1, mlp_b1, mlp_w2, mlp_b2):
    return _impl(64, 64, x, t, norm, flat_idx, conv_w, ln1_g, ln1_b,
                 ln2_g, ln2_b, tmlp1_w, tmlp1_b, tmlp2_w, tmlp2_b,
                 mlp_w1, mlp_b1, mlp_w2, mlp_b2)


# speedup vs baseline: 2.8233x; 1.0168x over previous
import functools
import math

import jax
import jax.numpy as jnp
from jax import lax
from jax.experimental import pallas as pl
from jax.experimental.pallas import tpu as pltpu
from jax.experimental.pallas import tpu_sc as plsc

NC = 2
NS = 16


def _rup(v, m):
    return ((v + m - 1) // m) * m


def _gelu(v):
    return 0.5 * v * (1.0 + lax.erf(v * jnp.float32(0.7071067811865476)))


def _ln(v, g, b):
    m = jnp.mean(v, axis=-1, keepdims=True)
    c = v - m
    var = jnp.mean(c * c, axis=-1, keepdims=True)
    return c * lax.rsqrt(var + jnp.float32(1e-5)) * g + b


def _impl(H, W, x, t, norm, flat_idx, conv_w, ln1_g, ln1_b, ln2_g, ln2_b,
          tmlp1_w, tmlp1_b, tmlp2_w, tmlp2_b, mlp_w1, mlp_b1, mlp_w2, mlp_b2):
    N, C = x.shape
    B = t.shape[0]
    K = conv_w.shape[0]
    R = K // 2
    HW = H * W
    Wp = W + 2 * R
    acc_need = (H - 1) * Wp + W
    BLK = min(256, _rup(acc_need, 8))
    nblk = -(-acc_need // BLK)
    ACC_B = nblk * BLK
    omax = (K - 1) * (Wp + 1)
    PB = _rup(ACC_B + omax, 8)
    NB = N // B
    RB = 512 if NB % 512 == 0 else NB
    nrb = N // RB
    TRASH = B * PB
    GR = _rup(B * PB + 8, NS * 8)
    NCH = N // 128

    f32 = jnp.float32
    i32 = jnp.int32
    wky = conv_w.reshape(K, K * C, C).astype(jnp.bfloat16)
    vspec = pl.BlockSpec(memory_space=pltpu.VMEM)

    def a_body(x_r, t_r, vcol_r, v2d_r, ln1g_r, ln1b_r, t1w_r, t1b_r,
               h_r, si_r, gi_r):
        tg = _gelu(t_r[...])
        tt1 = jnp.dot(tg, t1w_r[...], preferred_element_type=f32) + t1b_r[...]
        ln1g = ln1g_r[...]
        ln1b = ln1b_r[...]
        for rb in range(nrb):
            sl = pl.ds(rb * RB, RB)
            bi = (rb * RB) // NB
            f = _ln(x_r[sl, :], ln1g, ln1b)
            h_r[sl, :] = f * (1.0 + tt1[bi:bi + 1, :C]) + tt1[bi:bi + 1, C:]

        vcol = vcol_r[...]
        iglob = lax.broadcasted_iota(i32, (N, 128), 0)
        lane = lax.broadcasted_iota(i32, (N, 128), 1)
        dup = jnp.zeros((N, 1), i32)
        for jc in range(N // 128):
            cj = v2d_r[jc:jc + 1, :]
            eq = (vcol == cj) & ((jc * 128 + lane) > iglob)
            dup = jnp.maximum(dup, jnp.max(eq.astype(i32), axis=1,
                                           keepdims=True))
        b = vcol // HW
        r = vcol % HW
        y = r // W
        xx = r % W
        si = jnp.where(dup > 0, TRASH, b * PB + (y + R) * Wp + xx + R)
        si_r[...] = si
        gi_r[...] = b * ACC_B + y * Wp + xx

    h, si, gi = pl.pallas_call(
        a_body,
        out_shape=[jax.ShapeDtypeStruct((N, C), f32),
                   jax.ShapeDtypeStruct((N, 1), i32),
                   jax.ShapeDtypeStruct((N, 1), i32)],
        in_specs=[vspec] * 8,
        out_specs=[vspec] * 3,
    )(x, t, flat_idx.astype(i32).reshape(N, 1),
      flat_idx.astype(i32).reshape(N // 128, 128),
      ln1_g.reshape(1, C), ln1_b.reshape(1, C),
      tmlp1_w, tmlp1_b.reshape(1, 2 * C))
    si = si.reshape(N)
    gi = gi.reshape(N)

    mesh = plsc.VectorSubcoreMesh(core_axis_name="c", subcore_axis_name="s")
    rows_per_tile = N // NS
    nchunk = rows_per_tile // 128
    zr = GR // NS

    @functools.partial(
        pl.kernel, mesh=mesh, out_type=(),
        scratch_types=[pltpu.VMEM((128,), i32),
                       pltpu.VMEM((128, C), f32),
                       pltpu.SemaphoreType.DMA],
    )
    def scat_k(h_h, si_h, grid_h, idx_v, rows_v, sem):
        wid = lax.axis_index("s") * NC + lax.axis_index("c")
        base = wid * (N // (NC * NS))
        pltpu.sync_copy(si_h.at[pl.ds(base, 128)], idx_v)
        pltpu.sync_copy(h_h.at[pl.ds(base, 128)], rows_v)
        pltpu.async_copy(rows_v, grid_h.at[idx_v], sem).wait()

    gref = jax.new_ref(jnp.zeros((GR, C), f32))
    scat_k(h, si, gref)
    grid = gref[...]

    def c_body(grid_r, w_r, acc_r):
        def conv_blk(m, c):
            b = m // nblk
            g0 = (m % nblk) * BLK

            def ky_step(ky, acc):
                base = b * PB + g0 + ky * Wp
                src = jnp.concatenate(
                    [grid_r[pl.ds(base + kx, BLK), :] for kx in range(K)],
                    axis=1).astype(jnp.bfloat16)
                return acc + jnp.dot(src, w_r[ky],
                                     preferred_element_type=f32)

            accv = lax.fori_loop(0, K, ky_step, jnp.zeros((BLK, C), f32))
            acc_r[pl.ds(b * ACC_B + g0, BLK), :] = accv
            return c

        lax.fori_loop(0, B * nblk, conv_blk, 0)

    acc = pl.pallas_call(
        c_body,
        out_shape=jax.ShapeDtypeStruct((B * ACC_B, C), f32),
        in_specs=[vspec] * 2,
        out_specs=vspec,
    )(grid, wky)

    @functools.partial(
        pl.kernel, mesh=mesh,
        out_type=jax.ShapeDtypeStruct((N, C), f32),
        scratch_types=[pltpu.VMEM((128,), i32),
                       pltpu.VMEM((128, C), f32),
                       pltpu.SemaphoreType.DMA],
    )
    def gath_k(acc_h, gi_h, out_h, idx_v, rows_v, sem):
        wid = lax.axis_index("s") * NC + lax.axis_index("c")
        base = wid * (N // (NC * NS))
        pltpu.sync_copy(gi_h.at[pl.ds(base, 128)], idx_v)
        pltpu.async_copy(acc_h.at[idx_v], rows_v, sem).wait()
        pltpu.sync_copy(rows_v, out_h.at[pl.ds(base, 128)])

    hc = gath_k(acc, gi)

    def e_body(x_r, hc_r, norm_r, t_r, ln2g_r, ln2b_r, t2w_r, t2b_r,
               m1w_r, m1b_r, m2w_r, m2b_r, out_r):
        tg = _gelu(t_r[...])
        tt2 = jnp.dot(tg, t2w_r[...], preferred_element_type=f32) + t2b_r[...]
        ln2g = ln2g_r[...]
        ln2b = ln2b_r[...]
        m1w = m1w_r[...]
        m1b = m1b_r[...]
        m2w = m2w_r[...]
        m2b = m2b_r[...]
        for rb in range(nrb):
            sl = pl.ds(rb * RB, RB)
            bi = (rb * RB) // NB
            x1 = x_r[sl, :] + hc_r[sl, :] / norm_r[sl, :]
            f2 = _ln(x1, ln2g, ln2b)
            h2 = f2 * (1.0 + tt2[bi:bi + 1, :C]) + tt2[bi:bi + 1, C:]
            hid = _gelu(jnp.dot(h2, m1w, preferred_element_type=f32) + m1b)
            o = jnp.dot(hid, m2w, preferred_element_type=f32) + m2b
            out_r[sl, :] = x1 + o

    return pl.pallas_call(
        e_body,
        out_shape=jax.ShapeDtypeStruct((N, C), f32),
        in_specs=[vspec] * 12,
        out_specs=vspec,
    )(x, hc, norm, t, ln2_g.reshape(1, C), ln2_b.reshape(1, C),
      tmlp2_w, tmlp2_b.reshape(1, 2 * C),
      mlp_w1, mlp_b1.reshape(1, 2 * C), mlp_w2, mlp_b2.reshape(1, C))


def kernel(x, t, norm, flat_idx, conv_w, ln1_g, ln1_b, ln2_g, ln2_b,
           tmlp1_w, tmlp1_b, tmlp2_w, tmlp2_b, mlp_w1, mlp_b1, mlp_w2, mlp_b2):
    return _impl(64, 64, x, t, norm, flat_idx, conv_w, ln1_g, ln1_b,
                 ln2_g, ln2_b, tmlp1_w, tmlp1_b, tmlp2_w, tmlp2_b,
                 mlp_w1, mlp_b1, mlp_w2, mlp_b2)
